# baseline (speedup 1.0000x reference)
import jax
import jax.numpy as jnp
from jax import lax
from jax.experimental import pallas as pl
from jax.experimental.pallas import tpu as pltpu
from jax.experimental.pallas import tpu_sc as plsc

_NV = 1000001
_D = 64
_B = 16384
_NW = 32
_RW = _B // _NW
_CB = 32
_NCH = _RW // _CB
_NBUF = 2

_CN_STRIP = 96
_CTX_STRIP = 128
_CN_SPC = (_CB * 6) // _CN_STRIP
_CTX_SPC = (_CB * 20) // _CTX_STRIP


def _sc_body(cn_idx, ctx_idx, center_hbm, context_hbm, out_hbm,
             idx_cn_v, idx_ctx_v, cn_buf, ctx_buf, score_buf, sems):
    wid = lax.axis_index("s") * 2 + lax.axis_index("c")
    row0 = wid * _RW
    cnst0 = wid * (_RW * 6)
    ctxst0 = wid * (_RW * 20)

    def issue(g, b):
        pltpu.sync_copy(cn_idx.at[pl.ds(cnst0 + g * (6 * _CB), 6 * _CB)],
                        idx_cn_v.at[b])
        pltpu.sync_copy(ctx_idx.at[pl.ds(ctxst0 + g * (20 * _CB), 20 * _CB)],
                        idx_ctx_v.at[b])
        for j in range(_CN_SPC):
            pltpu.async_copy(
                center_hbm.at[idx_cn_v.at[b, pl.ds(j * _CN_STRIP, _CN_STRIP)]],
                cn_buf.at[b, pl.ds(j * _CN_STRIP, _CN_STRIP)],
                sems.at[b])
        for j in range(_CTX_SPC):
            pltpu.async_copy(
                context_hbm.at[idx_ctx_v.at[b, pl.ds(j * _CTX_STRIP, _CTX_STRIP)]],
                ctx_buf.at[b, pl.ds(j * _CTX_STRIP, _CTX_STRIP)],
                sems.at[b])

    def drain(b):
        pltpu.make_async_copy(center_hbm.at[pl.ds(0, 6 * _CB)],
                              cn_buf.at[b], sems.at[b]).wait()
        pltpu.make_async_copy(context_hbm.at[pl.ds(0, 20 * _CB)],
                              ctx_buf.at[b], sems.at[b]).wait()

    lane = lax.iota(jnp.int32, 16)
    lane15 = lane == 15

    def compute(g, b):
        def row_body(r, carry):
            cbase = r * 20
            acc = [ctx_buf[b, cbase, pl.ds(d * 16, 16)] for d in range(4)]
            for t in range(1, 20):
                for d in range(4):
                    acc[d] = acc[d] + ctx_buf[b, cbase + t, pl.ds(d * 16, 16)]
            nbase = r * 6
            for k in range(6):
                v = cn_buf[b, nbase + k, pl.ds(0, 16)] * acc[0]
                for d in range(1, 4):
                    v = v + cn_buf[b, nbase + k, pl.ds(d * 16, 16)] * acc[d]
                iv = jnp.full((16,), r * 6 + k, jnp.int32)
                plsc.store_scatter(score_buf, [iv], plsc.cumsum(v), mask=lane15)
            return carry

        lax.fori_loop(0, _CB, row_body, 0)
        pltpu.sync_copy(score_buf,
                        out_hbm.at[pl.ds((row0 + g * _CB) * 6, _CB * 6)])

    issue(0, 0)
    issue(1, 1)

    def pair_body(i, carry):
        for b in range(_NBUF):
            g = i * _NBUF + b
            drain(b)
            compute(g, b)

            @pl.when(g + _NBUF < _NCH)
            def _():
                issue(g + _NBUF, b)

        return carry

    lax.fori_loop(0, _NCH // _NBUF, pair_body, 0)


def kernel(x, center_table, context_table):
    xm = (x + _NV) % _NV
    cn_idx = xm[:, :6].reshape(_B * 6)
    ctx_idx = xm[:, 6:].reshape(_B * 20)

    mesh = plsc.VectorSubcoreMesh(core_axis_name="c", subcore_axis_name="s")
    run = pl.kernel(
        _sc_body,
        out_type=jax.ShapeDtypeStruct((_B * 6,), jnp.float32),
        mesh=mesh,
        compiler_params=pltpu.CompilerParams(use_tc_tiling_on_sc=False,
                                             needs_layout_passes=False),
        scratch_types=[
            pltpu.VMEM((_NBUF, 6 * _CB), jnp.int32),
            pltpu.VMEM((_NBUF, 20 * _CB), jnp.int32),
            pltpu.VMEM((_NBUF, 6 * _CB, _D), jnp.float32),
            pltpu.VMEM((_NBUF, 20 * _CB, _D), jnp.float32),
            pltpu.VMEM((_CB * 6,), jnp.float32),
            pltpu.SemaphoreType.DMA((_NBUF,)),
        ],
    )
    out = run(cn_idx, ctx_idx, center_table, context_table).reshape(_B, 6)
    return (out[:, :1], out[:, 1:])
